# Initial kernel scaffold; baseline (speedup 1.0000x reference)
import jax
import jax.numpy as jnp
from jax.experimental import pallas as pl


def kernel(idx, targets, table):
    raise NotImplementedError("write your pallas kernel here")


# speedup vs baseline: 1.4862x; 1.4862x over previous
import jax
import jax.numpy as jnp
from jax import lax
from jax.experimental import pallas as pl
from jax.experimental.pallas import tpu as pltpu
from jax.experimental.pallas import tpu_sc as plsc

VOCAB = 1000
LSE_PAD = 1024
NC, NS, L = 2, 16, 16
NW = NC * NS
CHUNK = 32


def _lse_body(table_ref, out_ref):
    x = table_ref[...]
    m = jnp.max(x, axis=1)
    s = jnp.sum(jnp.exp(x - m[:, None]), axis=1)
    out_ref[...] = m + jnp.log(s)


def _row_lse(table):
    return pl.pallas_call(
        _lse_body,
        out_shape=jax.ShapeDtypeStruct((VOCAB,), jnp.float32),
    )(table)


def _sc_body(table, idxf, tgtf, lse, out_logits, out_part,
             idx_v, tgt_v, lse_v, rows_v, acc_v, sem):
    n = out_logits.shape[0]
    rw = n // NW
    steps = rw // CHUNK
    wid = lax.axis_index("s") * NC + lax.axis_index("c")
    base = wid * rw
    pltpu.sync_copy(idxf.at[pl.ds(base, rw)], idx_v)
    pltpu.sync_copy(tgtf.at[pl.ds(base, rw)], tgt_v)
    pltpu.sync_copy(lse, lse_v)
    acc_v[...] = jnp.zeros((L,), jnp.float32)

    @pl.loop(0, steps)
    def _step(g):
        off = pl.multiple_of(g * CHUNK, CHUNK)
        pltpu.async_copy(table.at[idx_v.at[pl.ds(off, CHUNK)]], rows_v, sem).wait()
        for k in range(CHUNK // L):
            iv = idx_v[pl.ds(off + k * L, L)]
            tv = tgt_v[pl.ds(off + k * L, L)]
            rloc = lax.iota(jnp.int32, L) + (k * L)
            tvals = plsc.load_gather(rows_v, [rloc, tv])
            lvals = plsc.load_gather(lse_v, [iv])
            acc_v[...] = acc_v[...] + (lvals - tvals)
        pltpu.sync_copy(rows_v, out_logits.at[pl.ds(base + off, CHUNK)])

    pltpu.sync_copy(acc_v, out_part.at[wid])


def kernel(idx, targets, table):
    b, t = idx.shape
    n = b * t
    idxf = idx.reshape(n).astype(jnp.int32)
    tgtf = targets.reshape(n).astype(jnp.int32)
    lse = _row_lse(table)
    lse_pad = jnp.pad(lse, (0, LSE_PAD - VOCAB))
    rw = n // NW
    mesh = plsc.VectorSubcoreMesh(core_axis_name="c", subcore_axis_name="s")
    sc = pl.kernel(
        _sc_body,
        out_type=(jax.ShapeDtypeStruct((n, VOCAB), jnp.float32),
                  jax.ShapeDtypeStruct((NW, L), jnp.float32)),
        mesh=mesh,
        compiler_params=pltpu.CompilerParams(use_tc_tiling_on_sc=False,
                                             needs_layout_passes=False),
        scratch_types=[
            pltpu.VMEM((rw,), jnp.int32),
            pltpu.VMEM((rw,), jnp.int32),
            pltpu.VMEM((LSE_PAD,), jnp.float32),
            pltpu.VMEM((CHUNK, VOCAB), jnp.float32),
            pltpu.VMEM((L,), jnp.float32),
            pltpu.SemaphoreType.DMA,
        ],
    )
    logits_flat, part = sc(table, idxf, tgtf, lse_pad)
    logits = logits_flat.reshape(b, t, VOCAB)
    loss = jnp.sum(part) / n
    return (logits, loss)
